# baseline (speedup 1.0000x reference)
import functools

import jax
import jax.numpy as jnp
from jax import lax
from jax.experimental import pallas as pl
from jax.experimental.pallas import tpu as pltpu

_VMEM_LIMIT = 56 * 1024 * 1024
_BF = jnp.bfloat16
_F32 = jnp.float32


def _nn_up(x, s):
    if s == 1:
        return x
    h, w, c = x.shape
    x = jnp.broadcast_to(x[:, :, None, :], (h, w, s, c)).reshape(h, w * s, c)
    x = jnp.broadcast_to(x[:, None, :, :], (h, s, w * s, c)).reshape(h * s, w * s, c)
    return x


def _latdot(x, w_ref):
    return lax.dot_general(x.astype(_BF), w_ref[...], (((0,), (0,)), ((), ())),
                           preferred_element_type=_F32)


def _lat_top_kernel(x_ref, w_ref, o_ref):
    o_ref[0] = _latdot(x_ref[0], w_ref).astype(o_ref.dtype)


def _lat_top(x_flat, wt, tt):
    N, Cin, HW = x_flat.shape
    Cout = wt.shape[1]
    return pl.pallas_call(
        _lat_top_kernel,
        out_shape=jax.ShapeDtypeStruct((N, HW, Cout), _BF),
        grid=(N, HW // tt),
        in_specs=[pl.BlockSpec((1, Cin, tt), lambda n, t: (n, 0, t)),
                  pl.BlockSpec((Cin, Cout), lambda n, t: (0, 0))],
        out_specs=pl.BlockSpec((1, tt, Cout), lambda n, t: (n, t, 0)),
        compiler_params=pltpu.CompilerParams(
            dimension_semantics=("parallel", "parallel"),
            vmem_limit_bytes=_VMEM_LIMIT),
        cost_estimate=pl.CostEstimate(
            flops=2 * N * HW * Cin * Cout, transcendentals=0,
            bytes_accessed=4 * N * Cin * HW + 2 * N * HW * Cout),
    )(x_flat, wt)


def _mega_kernel(c4b, c3b, c2b, c4a, c3a, c2a, t5p, t5c, t5n,
                 w4_ref, w3_ref, w2_ref, wstk_ref, o_ref,
                 h4, h3, h2, t5a, v4, v3, v2, s5, s4, s3, s2, sems,
                 *, W2):
    n = pl.program_id(0)
    hb = pl.program_id(1)
    nblk = pl.num_programs(1)
    W3, W4, W5 = W2 // 2, W2 // 4, W2 // 8
    th2, th3, th4 = 16, 8, 4
    H2 = nblk * th2

    copies = []
    for i, (xa, hbuf, th, W, Hl) in enumerate(
            ((c4a, h4, th4, W4, H2 // 4), (c3a, h3, th3, W3, H2 // 2),
             (c2a, h2, th2, W2, H2))):
        if W >= 128:
            top = pl.multiple_of(jnp.maximum(hb * th - 1, 0) * W, 128)
            bot = pl.multiple_of(jnp.minimum(hb * th + th, Hl - 1) * W, 128)
            fw = W
        else:
            top = pl.multiple_of(jnp.maximum(hb * th - 2, 0) * W, 128)
            bot = pl.multiple_of(jnp.minimum(hb * th + th, Hl - 2) * W, 128)
            fw = 2 * W
        ct = pltpu.make_async_copy(xa.at[n, :, pl.ds(top, fw)],
                                   hbuf.at[:, pl.ds(0, fw)], sems.at[2 * i])
        cb = pltpu.make_async_copy(xa.at[n, :, pl.ds(bot, fw)],
                                   hbuf.at[:, pl.ds(fw, fw)], sems.at[2 * i + 1])
        ct.start()
        cb.start()
        copies.append((ct, cb))

    def conv_branch(idx, v, th, W, cast):
        Br = (s5, s4, s3, s2)[idx]
        T = th * W
        z = jnp.zeros((W, 256), _BF)

        def bf(a):
            return a.astype(_BF) if cast else a
        Br[:, 256:512] = bf(v[1:th + 1].reshape(T, 256))
        Br[W:, 0:256] = bf(v[1:th].reshape(T - W, 256))
        Br[0:W, 0:256] = jnp.where(hb == 0, z, bf(v[0].reshape(W, 256)))
        Br[:T - W, 512:768] = bf(v[2:th + 1].reshape(T - W, 256))
        Br[T - W:, 512:768] = jnp.where(hb == nblk - 1, z,
                                        bf(v[th + 1].reshape(W, 256)))
        S = jnp.dot(Br[...], wstk_ref[idx],
                    preferred_element_type=_F32).reshape(th, W, 384)
        zc = jnp.zeros((th, 1, 128), _F32)
        return (S[:, :, 128:256]
                + jnp.concatenate([zc, S[:, :-1, 0:128]], axis=1)
                + jnp.concatenate([S[:, 1:, 256:384], zc], axis=1))

    def store_branch(idx, y, scale):
        up = _nn_up(y, scale)
        yt = jnp.transpose(up.reshape(th2 * W2, 128))[0:64]
        o_ref[0, 64 * idx:64 * (idx + 1), :] = yt.astype(o_ref.dtype)

    t5a[1:3] = t5c[0]
    t5a[0:1] = t5p[0, 1:2]
    t5a[3:4] = t5n[0, 0:1]
    store_branch(0, conv_branch(0, t5a, 2, W5, cast=False), 8)

    def level(i, hbuf, cb_ref, w_ref, vdst, th, W, up_src):
        ct, cpb = copies[i]
        ct.wait()
        cpb.wait()
        lat_h = _latdot(hbuf[...], w_ref).reshape(-1, W, 256)
        ti, bi = (1, 2) if W < 128 else (0, 1)
        lat_c = _latdot(cb_ref[0], w_ref).reshape(th, W, 256)
        upf = _nn_up(up_src, 2)
        vdst[0:1] = lat_h[ti:ti + 1] + upf[1:2]
        vdst[1:th + 1] = lat_c + upf[2:th + 2]
        vdst[th + 1:th + 2] = lat_h[bi:bi + 1] + upf[th + 2:th + 3]

    level(0, h4, c4b, w4_ref, v4, th4, W4, t5a[...].astype(_F32))
    store_branch(1, conv_branch(1, v4, th4, W4, cast=True), 4)
    level(1, h3, c3b, w3_ref, v3, th3, W3, v4[...])
    store_branch(2, conv_branch(2, v3, th3, W3, cast=True), 2)
    level(2, h2, c2b, w2_ref, v2, th2, W2, v3[...])
    store_branch(3, conv_branch(3, v2, th2, W2, cast=True), 1)


def _mega(c4f, c3f, c2f, t5, w4t, w3t, w2t, wstk):
    N = c2f.shape[0]
    W5, H5 = t5.shape[2], t5.shape[1]
    W2, H2 = 8 * W5, 8 * H5
    W3, W4 = W2 // 2, W2 // 4
    nblk = H2 // 16
    flops = (2 * N * 256 * (H2 * W2 * 256 + (H2 // 2) * (W2 // 2) * 512
                            + (H2 // 4) * (W2 // 4) * 1024)
             + sum(2 * 9 * N * (H2 // s) * (W2 // s) * 256 * 64
                   for s in (8, 4, 2, 1)))
    bytes_accessed = 4 * (N * 1024 * (H2 // 4) * (W2 // 4)
                          + N * 512 * (H2 // 2) * (W2 // 2)
                          + N * 256 * H2 * W2 + N * 256 * H2 * W2)
    kern = functools.partial(_mega_kernel, W2=W2)
    return pl.pallas_call(
        kern,
        out_shape=jax.ShapeDtypeStruct((N, 256, H2 * W2), _F32),
        grid=(N, nblk),
        in_specs=[
            pl.BlockSpec((1, 1024, 4 * W4), lambda n, h: (n, 0, h)),
            pl.BlockSpec((1, 512, 8 * W3), lambda n, h: (n, 0, h)),
            pl.BlockSpec((1, 256, 16 * W2), lambda n, h: (n, 0, h)),
            pl.BlockSpec(memory_space=pl.ANY),
            pl.BlockSpec(memory_space=pl.ANY),
            pl.BlockSpec(memory_space=pl.ANY),
            pl.BlockSpec((1, 2, W5, 256), lambda n, h: (n, jnp.maximum(h - 1, 0), 0, 0)),
            pl.BlockSpec((1, 2, W5, 256), lambda n, h: (n, h, 0, 0)),
            pl.BlockSpec((1, 2, W5, 256),
                         lambda n, h, nb=nblk: (n, jnp.minimum(h + 1, nb - 1), 0, 0)),
            pl.BlockSpec((1024, 256), lambda n, h: (0, 0)),
            pl.BlockSpec((512, 256), lambda n, h: (0, 0)),
            pl.BlockSpec((256, 256), lambda n, h: (0, 0)),
            pl.BlockSpec((4, 768, 384), lambda n, h: (0, 0, 0)),
        ],
        out_specs=pl.BlockSpec((1, 256, 16 * W2), lambda n, h: (n, 0, h)),
        scratch_shapes=[
            pltpu.VMEM((1024, (4 if W4 < 128 else 2) * W4), _F32),
            pltpu.VMEM((512, (4 if W3 < 128 else 2) * W3), _F32),
            pltpu.VMEM((256, (4 if W2 < 128 else 2) * W2), _F32),
            pltpu.VMEM((4, W5, 256), _BF),
            pltpu.VMEM((6, W4, 256), _F32),
            pltpu.VMEM((10, W3, 256), _F32),
            pltpu.VMEM((18, W2, 256), _F32),
            pltpu.VMEM((2 * W5, 768), _BF),
            pltpu.VMEM((4 * W4, 768), _BF),
            pltpu.VMEM((8 * W3, 768), _BF),
            pltpu.VMEM((16 * W2, 768), _BF),
            pltpu.SemaphoreType.DMA((6,)),
        ],
        compiler_params=pltpu.CompilerParams(
            dimension_semantics=("parallel", "parallel"),
            vmem_limit_bytes=_VMEM_LIMIT),
        cost_estimate=pl.CostEstimate(
            flops=flops, transcendentals=0, bytes_accessed=bytes_accessed),
    )(c4f, c3f, c2f, c4f, c3f, c2f, t5, t5, t5, w4t, w3t, w2t, wstk)


def _mk_conv_w(p):
    wt = jnp.transpose(p, (2, 3, 1, 0))
    wt = jnp.pad(wt, ((0, 0), (0, 0), (0, 0), (0, 64)))
    return jnp.transpose(wt, (0, 2, 1, 3)).reshape(768, 384).astype(_BF)


def kernel(c2, c3, c4, c5, in2, in3, in4, in5, p5, p4, p3, p2):
    N = c2.shape[0]
    w5t = jnp.transpose(in5).astype(_BF)
    w4t = jnp.transpose(in4).astype(_BF)
    w3t = jnp.transpose(in3).astype(_BF)
    w2t = jnp.transpose(in2).astype(_BF)
    wstk = jnp.stack([_mk_conv_w(p5), _mk_conv_w(p4),
                      _mk_conv_w(p3), _mk_conv_w(p2)])

    c5f = c5.reshape(N, c5.shape[1], -1)
    c4f = c4.reshape(N, c4.shape[1], -1)
    c3f = c3.reshape(N, c3.shape[1], -1)
    c2f = c2.reshape(N, c2.shape[1], -1)

    t5 = _lat_top(c5f, w5t, min(512, c5f.shape[2]))
    t5 = t5.reshape(N, c5.shape[2], c5.shape[3], 256)

    fuse = _mega(c4f, c3f, c2f, t5, w4t, w3t, w2t, wstk)
    return fuse.reshape(N, 256, c2.shape[2], c2.shape[3])


# speedup vs baseline: 1.0516x; 1.0516x over previous
import jax
import jax.numpy as jnp
from jax import lax
from jax.experimental import pallas as pl
from jax.experimental.pallas import tpu as pltpu

_VMEM_LIMIT = 56 * 1024 * 1024
_BF = jnp.bfloat16
_F32 = jnp.float32


def _nn_up(x, s):
    if s == 1:
        return x
    h, w, c = x.shape
    x = jnp.broadcast_to(x[:, :, None, :], (h, w, s, c)).reshape(h, w * s, c)
    x = jnp.broadcast_to(x[:, None, :, :], (h, s, w * s, c)).reshape(h * s, w * s, c)
    return x


def _latdot(x, w_ref):
    return lax.dot_general(x.astype(_BF), w_ref[...], (((0,), (0,)), ((), ())),
                           preferred_element_type=_F32)


def _lat_top_kernel(x_ref, w_ref, o_ref):
    o_ref[0] = _latdot(x_ref[0], w_ref).astype(o_ref.dtype)


def _lat_top(x_flat, wt, tt):
    N, Cin, HW = x_flat.shape
    Cout = wt.shape[1]
    return pl.pallas_call(
        _lat_top_kernel,
        out_shape=jax.ShapeDtypeStruct((N, HW, Cout), _BF),
        grid=(N, HW // tt),
        in_specs=[pl.BlockSpec((1, Cin, tt), lambda n, t: (n, 0, t)),
                  pl.BlockSpec((Cin, Cout), lambda n, t: (0, 0))],
        out_specs=pl.BlockSpec((1, tt, Cout), lambda n, t: (n, t, 0)),
        compiler_params=pltpu.CompilerParams(
            dimension_semantics=("parallel", "parallel"),
            vmem_limit_bytes=_VMEM_LIMIT),
        cost_estimate=pl.CostEstimate(
            flops=2 * N * HW * Cin * Cout, transcendentals=0,
            bytes_accessed=4 * N * Cin * HW + 2 * N * HW * Cout),
    )(x_flat, wt)


def _lat_chain_kernel(c4_ref, c3_ref, c2_ref, t5_ref, w4_ref, w3_ref, w2_ref,
                      o4_ref, o3_ref, o2_ref):
    v4 = _latdot(c4_ref[0], w4_ref).reshape(o4_ref.shape[1:]) + _nn_up(
        t5_ref[0].astype(_F32), 2)
    o4_ref[0] = v4.astype(o4_ref.dtype)
    v3 = _latdot(c3_ref[0], w3_ref).reshape(o3_ref.shape[1:]) + _nn_up(v4, 2)
    o3_ref[0] = v3.astype(o3_ref.dtype)
    v2 = _latdot(c2_ref[0], w2_ref).reshape(o2_ref.shape[1:]) + _nn_up(v3, 2)
    o2_ref[0] = v2.astype(o2_ref.dtype)


def _lat_chain(c4f, c3f, c2f, t5, w4t, w3t, w2t, *, th2=16):
    N = c2f.shape[0]
    W2 = t5.shape[2] * 8
    H2 = c2f.shape[2] // W2
    H4, W4, H3, W3 = H2 // 4, W2 // 4, H2 // 2, W2 // 2
    th4, th3 = th2 // 4, th2 // 2
    flops = 2 * N * 256 * (H2 * W2 * 256 + H3 * W3 * 512 + H4 * W4 * 1024)
    out_shapes = [jax.ShapeDtypeStruct((N, H4, W4, 256), _BF),
                  jax.ShapeDtypeStruct((N, H3, W3, 256), _BF),
                  jax.ShapeDtypeStruct((N, H2, W2, 256), _BF)]
    return pl.pallas_call(
        _lat_chain_kernel,
        out_shape=out_shapes,
        grid=(N, H2 // th2),
        in_specs=[
            pl.BlockSpec((1, 1024, th4 * W4), lambda n, h: (n, 0, h)),
            pl.BlockSpec((1, 512, th3 * W3), lambda n, h: (n, 0, h)),
            pl.BlockSpec((1, 256, th2 * W2), lambda n, h: (n, 0, h)),
            pl.BlockSpec((1, th4 // 2, W4 // 2, 256), lambda n, h: (n, h, 0, 0)),
            pl.BlockSpec((1024, 256), lambda n, h: (0, 0)),
            pl.BlockSpec((512, 256), lambda n, h: (0, 0)),
            pl.BlockSpec((256, 256), lambda n, h: (0, 0)),
        ],
        out_specs=[
            pl.BlockSpec((1, th4, W4, 256), lambda n, h: (n, h, 0, 0)),
            pl.BlockSpec((1, th3, W3, 256), lambda n, h: (n, h, 0, 0)),
            pl.BlockSpec((1, th2, W2, 256), lambda n, h: (n, h, 0, 0)),
        ],
        compiler_params=pltpu.CompilerParams(
            dimension_semantics=("parallel", "parallel"),
            vmem_limit_bytes=_VMEM_LIMIT),
        cost_estimate=pl.CostEstimate(
            flops=flops, transcendentals=0,
            bytes_accessed=4 * (N * 1024 * H4 * W4 + N * 512 * H3 * W3
                                + N * 256 * H2 * W2)
            + 2 * N * 256 * (H4 * W4 + H3 * W3 + H2 * W2)),
    )(c4f, c3f, c2f, t5, w4t, w3t, w2t)


_SCALES = (8, 4, 2, 1)
_TH2 = 16


def _fuse_kernel(c5b, c4b, c3b, c2b, x5, x4, x3, x2, w_ref, o_ref,
                 h5, h4, h3, h2, s5, s4, s3, s2, sems):
    n = pl.program_id(0)
    hb = pl.program_id(1)
    nblk = pl.num_programs(1)
    ctr = (c5b, c4b, c3b, c2b)
    halos = (h5, h4, h3, h2)
    stks = (s5, s4, s3, s2)
    srcs = (x5, x4, x3, x2)
    copies = []
    for idx in range(4):
        th = _TH2 // _SCALES[idx]
        xr, hr = srcs[idx], halos[idx]
        H = xr.shape[1]
        r0 = hb * th
        ct = pltpu.make_async_copy(xr.at[n, pl.ds(jnp.maximum(r0 - 1, 0), 1)],
                                   hr.at[pl.ds(0, 1)], sems.at[2 * idx])
        cb = pltpu.make_async_copy(xr.at[n, pl.ds(jnp.minimum(r0 + th, H - 1), 1)],
                                   hr.at[pl.ds(1, 1)], sems.at[2 * idx + 1])
        ct.start()
        cb.start()
        copies.append((ct, cb))
    for idx in range(4):
        th = _TH2 // _SCALES[idx]
        xc, Br = ctr[idx], stks[idx]
        W = xc.shape[2]
        T = th * W
        Br[:, 256:512] = xc[0].reshape(T, 256)
        Br[W:, 0:256] = xc[0, 0:th - 1].reshape(T - W, 256)
        Br[:T - W, 512:768] = xc[0, 1:th].reshape(T - W, 256)
    for idx in range(4):
        scale = _SCALES[idx]
        th = _TH2 // scale
        hr, Br = halos[idx], stks[idx]
        W = srcs[idx].shape[2]
        T = th * W
        ct, cb = copies[idx]
        ct.wait()
        cb.wait()
        z = jnp.zeros((W, 256), _BF)
        Br[0:W, 0:256] = jnp.where(hb == 0, z, hr[0].reshape(W, 256))
        Br[T - W:, 512:768] = jnp.where(hb == nblk - 1, z, hr[1].reshape(W, 256))
        S = jnp.dot(Br[...], w_ref[idx],
                    preferred_element_type=_F32).reshape(th, W, 384)
        zc = jnp.zeros((th, 1, 128), _F32)
        y = (S[:, :, 128:256]
             + jnp.concatenate([zc, S[:, :-1, 0:128]], axis=1)
             + jnp.concatenate([S[:, 1:, 256:384], zc], axis=1))
        up = _nn_up(y, scale)
        hw = up.shape[0] * up.shape[1]
        yt = jnp.transpose(up.reshape(hw, 128))[0:64]
        o_ref[0, 64 * idx:64 * (idx + 1), :] = yt.astype(o_ref.dtype)


def _fused_convs_concat(in5, out4, out3, out2, wstk):
    N, H2, W2, Cb = out2.shape[0], out2.shape[1], out2.shape[2], 64
    flops = sum(2 * 9 * N * (H2 // s) * (W2 // s) * 256 * 64 for s in _SCALES)
    halos = [pltpu.VMEM((2, W2 // s, 256), _BF) for s in _SCALES]
    stks = [pltpu.VMEM(((_TH2 // s) * (W2 // s), 768), _BF) for s in _SCALES]
    ctr_specs = [
        pl.BlockSpec((1, _TH2 // s, W2 // s, 256), lambda n, h: (n, h, 0, 0))
        for s in _SCALES]
    return pl.pallas_call(
        _fuse_kernel,
        out_shape=jax.ShapeDtypeStruct((N, 4 * Cb, H2 * W2), _F32),
        grid=(N, H2 // _TH2),
        in_specs=ctr_specs + [
            pl.BlockSpec(memory_space=pl.ANY),
            pl.BlockSpec(memory_space=pl.ANY),
            pl.BlockSpec(memory_space=pl.ANY),
            pl.BlockSpec(memory_space=pl.ANY),
            pl.BlockSpec((4, 768, 384), lambda n, h: (0, 0, 0))],
        out_specs=pl.BlockSpec((1, 4 * Cb, _TH2 * W2), lambda n, h: (n, 0, h)),
        scratch_shapes=halos + stks + [pltpu.SemaphoreType.DMA((8,))],
        compiler_params=pltpu.CompilerParams(
            dimension_semantics=("parallel", "parallel"),
            vmem_limit_bytes=_VMEM_LIMIT),
        cost_estimate=pl.CostEstimate(
            flops=flops, transcendentals=0,
            bytes_accessed=4 * N * H2 * W2 * 4 * Cb
            + 2 * N * (H2 * W2 + 3 * (H2 // 2) * (W2 // 2)) * 256),
    )(in5, out4, out3, out2, in5, out4, out3, out2, wstk)


def _mk_conv_w(p):
    wt = jnp.transpose(p, (2, 3, 1, 0))
    wt = jnp.pad(wt, ((0, 0), (0, 0), (0, 0), (0, 64)))
    return jnp.transpose(wt, (0, 2, 1, 3)).reshape(768, 384).astype(_BF)


def kernel(c2, c3, c4, c5, in2, in3, in4, in5, p5, p4, p3, p2):
    N = c2.shape[0]
    w5t = jnp.transpose(in5).astype(_BF)
    w4t = jnp.transpose(in4).astype(_BF)
    w3t = jnp.transpose(in3).astype(_BF)
    w2t = jnp.transpose(in2).astype(_BF)
    wstk = jnp.stack([_mk_conv_w(p5), _mk_conv_w(p4),
                      _mk_conv_w(p3), _mk_conv_w(p2)])

    c5f = c5.reshape(N, c5.shape[1], -1)
    c4f = c4.reshape(N, c4.shape[1], -1)
    c3f = c3.reshape(N, c3.shape[1], -1)
    c2f = c2.reshape(N, c2.shape[1], -1)

    t5 = _lat_top(c5f, w5t, min(512, c5f.shape[2]))
    t5 = t5.reshape(N, c5.shape[2], c5.shape[3], 256)
    o4, o3, o2 = _lat_chain(c4f, c3f, c2f, t5, w4t, w3t, w2t,
                            th2=min(16, c2.shape[2]))

    fuse = _fused_convs_concat(t5, o4, o3, o2, wstk)
    return fuse.reshape(N, 256, c2.shape[2], c2.shape[3])


# speedup vs baseline: 1.2946x; 1.2310x over previous
import jax
import jax.numpy as jnp
from jax import lax
from jax.experimental import pallas as pl
from jax.experimental.pallas import tpu as pltpu

_VMEM_LIMIT = 56 * 1024 * 1024
_BF = jnp.bfloat16
_F32 = jnp.float32


def _nn_up(x, s):
    if s == 1:
        return x
    h, w, c = x.shape
    x = jnp.broadcast_to(x[:, :, None, :], (h, w, s, c)).reshape(h, w * s, c)
    x = jnp.broadcast_to(x[:, None, :, :], (h, s, w * s, c)).reshape(h * s, w * s, c)
    return x


def _latdot(x, w_ref):
    return lax.dot_general(x.astype(_BF), w_ref[...], (((0,), (0,)), ((), ())),
                           preferred_element_type=_F32)


def _lat_top_kernel(x_ref, w_ref, o_ref):
    o_ref[0] = _latdot(x_ref[0], w_ref).astype(o_ref.dtype)


def _lat_top(x_flat, wt, tt):
    N, Cin, HW = x_flat.shape
    Cout = wt.shape[1]
    return pl.pallas_call(
        _lat_top_kernel,
        out_shape=jax.ShapeDtypeStruct((N, HW, Cout), _BF),
        grid=(N, HW // tt),
        in_specs=[pl.BlockSpec((1, Cin, tt), lambda n, t: (n, 0, t)),
                  pl.BlockSpec((Cin, Cout), lambda n, t: (0, 0))],
        out_specs=pl.BlockSpec((1, tt, Cout), lambda n, t: (n, t, 0)),
        compiler_params=pltpu.CompilerParams(
            dimension_semantics=("parallel", "parallel"),
            vmem_limit_bytes=_VMEM_LIMIT),
        cost_estimate=pl.CostEstimate(
            flops=2 * N * HW * Cin * Cout, transcendentals=0,
            bytes_accessed=4 * N * Cin * HW + 2 * N * HW * Cout),
    )(x_flat, wt)


def _lat_chain_kernel(c4_ref, c3_ref, c2_ref, t5_ref, w4_ref, w3_ref, w2_ref,
                      o4_ref, o3_ref, o2_ref, ho4_ref, ho3_ref, ho2_ref):
    v4 = _latdot(c4_ref[0], w4_ref).reshape(o4_ref.shape[1:]) + _nn_up(
        t5_ref[0].astype(_F32), 2)
    o4_ref[0] = v4.astype(o4_ref.dtype)
    ho4_ref[0, 0, 0] = v4[0].astype(ho4_ref.dtype)
    ho4_ref[0, 0, 1] = v4[-1].astype(ho4_ref.dtype)
    v3 = _latdot(c3_ref[0], w3_ref).reshape(o3_ref.shape[1:]) + _nn_up(v4, 2)
    o3_ref[0] = v3.astype(o3_ref.dtype)
    ho3_ref[0, 0, 0] = v3[0].astype(ho3_ref.dtype)
    ho3_ref[0, 0, 1] = v3[-1].astype(ho3_ref.dtype)
    v2 = _latdot(c2_ref[0], w2_ref).reshape(o2_ref.shape[1:]) + _nn_up(v3, 2)
    o2_ref[0] = v2.astype(o2_ref.dtype)
    ho2_ref[0, 0, 0] = v2[0].astype(ho2_ref.dtype)
    ho2_ref[0, 0, 1] = v2[-1].astype(ho2_ref.dtype)


def _lat_chain(c4f, c3f, c2f, t5, w4t, w3t, w2t, *, th2=16):
    N = c2f.shape[0]
    W2 = t5.shape[2] * 8
    H2 = c2f.shape[2] // W2
    H4, W4, H3, W3 = H2 // 4, W2 // 4, H2 // 2, W2 // 2
    th4, th3 = th2 // 4, th2 // 2
    flops = 2 * N * 256 * (H2 * W2 * 256 + H3 * W3 * 512 + H4 * W4 * 1024)
    nblk = H2 // th2
    out_shapes = [jax.ShapeDtypeStruct((N, H4, W4, 256), _BF),
                  jax.ShapeDtypeStruct((N, H3, W3, 256), _BF),
                  jax.ShapeDtypeStruct((N, H2, W2, 256), _BF),
                  jax.ShapeDtypeStruct((N, nblk, 2, W4, 256), _BF),
                  jax.ShapeDtypeStruct((N, nblk, 2, W3, 256), _BF),
                  jax.ShapeDtypeStruct((N, nblk, 2, W2, 256), _BF)]
    return pl.pallas_call(
        _lat_chain_kernel,
        out_shape=out_shapes,
        grid=(N, H2 // th2),
        in_specs=[
            pl.BlockSpec((1, 1024, th4 * W4), lambda n, h: (n, 0, h)),
            pl.BlockSpec((1, 512, th3 * W3), lambda n, h: (n, 0, h)),
            pl.BlockSpec((1, 256, th2 * W2), lambda n, h: (n, 0, h)),
            pl.BlockSpec((1, th4 // 2, W4 // 2, 256), lambda n, h: (n, h, 0, 0)),
            pl.BlockSpec((1024, 256), lambda n, h: (0, 0)),
            pl.BlockSpec((512, 256), lambda n, h: (0, 0)),
            pl.BlockSpec((256, 256), lambda n, h: (0, 0)),
        ],
        out_specs=[
            pl.BlockSpec((1, th4, W4, 256), lambda n, h: (n, h, 0, 0)),
            pl.BlockSpec((1, th3, W3, 256), lambda n, h: (n, h, 0, 0)),
            pl.BlockSpec((1, th2, W2, 256), lambda n, h: (n, h, 0, 0)),
            pl.BlockSpec((1, 1, 2, W4, 256), lambda n, h: (n, h, 0, 0, 0)),
            pl.BlockSpec((1, 1, 2, W3, 256), lambda n, h: (n, h, 0, 0, 0)),
            pl.BlockSpec((1, 1, 2, W2, 256), lambda n, h: (n, h, 0, 0, 0)),
        ],
        compiler_params=pltpu.CompilerParams(
            dimension_semantics=("parallel", "parallel"),
            vmem_limit_bytes=_VMEM_LIMIT),
        cost_estimate=pl.CostEstimate(
            flops=flops, transcendentals=0,
            bytes_accessed=4 * (N * 1024 * H4 * W4 + N * 512 * H3 * W3
                                + N * 256 * H2 * W2)
            + 2 * N * 256 * (H4 * W4 + H3 * W3 + H2 * W2)),
    )(c4f, c3f, c2f, t5, w4t, w3t, w2t)


_SCALES = (8, 4, 2, 1)
_TH2 = 16


def _fuse_kernel(t5p, t5c, t5n, c4c, h4p, h4n, c3c, h3p, h3n, c2c, h2p, h2n,
                 w_ref, o_ref, s5, s4, s3, s2):
    hb = pl.program_id(1)
    nblk = pl.num_programs(1)
    lvls = ((t5c, t5p, t5n), (c4c, h4p, h4n), (c3c, h3p, h3n), (c2c, h2p, h2n))
    stks = (s5, s4, s3, s2)
    for idx in range(4):
        scale = _SCALES[idx]
        th = _TH2 // scale
        xc, hp, hn = lvls[idx]
        Br = stks[idx]
        W = xc.shape[2]
        T = th * W
        if idx == 0:
            top = hp[0, th - 1]
            bot = hn[0, 0]
        else:
            top = hp[0, 0, 1]
            bot = hn[0, 0, 0]
        z = jnp.zeros((W, 256), _BF)
        Br[:, 256:512] = xc[0].reshape(T, 256)
        Br[W:, 0:256] = xc[0, 0:th - 1].reshape(T - W, 256)
        Br[:T - W, 512:768] = xc[0, 1:th].reshape(T - W, 256)
        Br[0:W, 0:256] = jnp.where(hb == 0, z, top.reshape(W, 256))
        Br[T - W:, 512:768] = jnp.where(hb == nblk - 1, z, bot.reshape(W, 256))
        S = jnp.dot(Br[...], w_ref[idx],
                    preferred_element_type=_F32).reshape(th, W, 384)
        zc = jnp.zeros((th, 1, 128), _F32)
        y = (S[:, :, 128:256]
             + jnp.concatenate([zc, S[:, :-1, 0:128]], axis=1)
             + jnp.concatenate([S[:, 1:, 256:384], zc], axis=1))
        up = _nn_up(y, scale)
        hw = up.shape[0] * up.shape[1]
        yt = jnp.transpose(up.reshape(hw, 128))[0:64]
        o_ref[0, 64 * idx:64 * (idx + 1), :] = yt.astype(o_ref.dtype)


def _fused_convs_concat(in5, out4, out3, out2, ho4, ho3, ho2, wstk):
    N, H2, W2, Cb = out2.shape[0], out2.shape[1], out2.shape[2], 64
    nblk = H2 // _TH2
    flops = sum(2 * 9 * N * (H2 // s) * (W2 // s) * 256 * 64 for s in _SCALES)
    stks = [pltpu.VMEM(((_TH2 // s) * (W2 // s), 768), _BF) for s in _SCALES]

    def prv(n, h):
        return (n, jnp.maximum(h - 1, 0), 0, 0)

    def nxt(n, h):
        return (n, jnp.minimum(h + 1, nblk - 1), 0, 0)

    def prv5(n, h):
        return (n, jnp.maximum(h - 1, 0), 0, 0, 0)

    def nxt5(n, h):
        return (n, jnp.minimum(h + 1, nblk - 1), 0, 0, 0)

    specs = [
        pl.BlockSpec((1, _TH2 // 8, W2 // 8, 256), prv),
        pl.BlockSpec((1, _TH2 // 8, W2 // 8, 256), lambda n, h: (n, h, 0, 0)),
        pl.BlockSpec((1, _TH2 // 8, W2 // 8, 256), nxt),
    ]
    for s, ho in ((4, ho4), (2, ho3), (1, ho2)):
        specs += [
            pl.BlockSpec((1, _TH2 // s, W2 // s, 256), lambda n, h: (n, h, 0, 0)),
            pl.BlockSpec((1, 1, 2, W2 // s, 256), prv5),
            pl.BlockSpec((1, 1, 2, W2 // s, 256), nxt5),
        ]
    specs.append(pl.BlockSpec((4, 768, 384), lambda n, h: (0, 0, 0)))
    return pl.pallas_call(
        _fuse_kernel,
        out_shape=jax.ShapeDtypeStruct((N, 4 * Cb, H2 * W2), _F32),
        grid=(N, nblk),
        in_specs=specs,
        out_specs=pl.BlockSpec((1, 4 * Cb, _TH2 * W2), lambda n, h: (n, 0, h)),
        scratch_shapes=stks,
        compiler_params=pltpu.CompilerParams(
            dimension_semantics=("parallel", "parallel"),
            vmem_limit_bytes=_VMEM_LIMIT),
        cost_estimate=pl.CostEstimate(
            flops=flops, transcendentals=0,
            bytes_accessed=4 * N * H2 * W2 * 4 * Cb
            + 2 * N * (H2 * W2 + 3 * (H2 // 2) * (W2 // 2)) * 256),
    )(in5, in5, in5, out4, ho4, ho4, out3, ho3, ho3, out2, ho2, ho2, wstk)


def _mk_conv_w(p):
    wt = jnp.transpose(p, (2, 3, 1, 0))
    wt = jnp.pad(wt, ((0, 0), (0, 0), (0, 0), (0, 64)))
    return jnp.transpose(wt, (0, 2, 1, 3)).reshape(768, 384).astype(_BF)


def kernel(c2, c3, c4, c5, in2, in3, in4, in5, p5, p4, p3, p2):
    N = c2.shape[0]
    w5t = jnp.transpose(in5).astype(_BF)
    w4t = jnp.transpose(in4).astype(_BF)
    w3t = jnp.transpose(in3).astype(_BF)
    w2t = jnp.transpose(in2).astype(_BF)
    wstk = jnp.stack([_mk_conv_w(p5), _mk_conv_w(p4),
                      _mk_conv_w(p3), _mk_conv_w(p2)])

    c5f = c5.reshape(N, c5.shape[1], -1)
    c4f = c4.reshape(N, c4.shape[1], -1)
    c3f = c3.reshape(N, c3.shape[1], -1)
    c2f = c2.reshape(N, c2.shape[1], -1)

    t5 = _lat_top(c5f, w5t, min(512, c5f.shape[2]))
    t5 = t5.reshape(N, c5.shape[2], c5.shape[3], 256)
    o4, o3, o2, ho4, ho3, ho2 = _lat_chain(c4f, c3f, c2f, t5, w4t, w3t, w2t,
                                           th2=min(16, c2.shape[2]))

    fuse = _fused_convs_concat(t5, o4, o3, o2, ho4, ho3, ho2, wstk)
    return fuse.reshape(N, 256, c2.shape[2], c2.shape[3])


# speedup vs baseline: 1.3014x; 1.0053x over previous
import jax
import jax.numpy as jnp
from jax import lax
from jax.experimental import pallas as pl
from jax.experimental.pallas import tpu as pltpu

_VMEM_LIMIT = 56 * 1024 * 1024
_BF = jnp.bfloat16
_F32 = jnp.float32


def _nn_up(x, s):
    if s == 1:
        return x
    h, w, c = x.shape
    x = jnp.broadcast_to(x[:, :, None, :], (h, w, s, c)).reshape(h, w * s, c)
    x = jnp.broadcast_to(x[:, None, :, :], (h, s, w * s, c)).reshape(h * s, w * s, c)
    return x


def _latdot(x, w):
    return lax.dot_general(x.astype(_BF), w, (((0,), (0,)), ((), ())),
                           preferred_element_type=_F32)


def _lat_top_kernel(x_ref, w_ref, o_ref):
    o_ref[0] = _latdot(x_ref[0], w_ref[...]).astype(o_ref.dtype)


def _lat_top(x_flat, wt, tt):
    N, Cin, HW = x_flat.shape
    Cout = wt.shape[1]
    return pl.pallas_call(
        _lat_top_kernel,
        out_shape=jax.ShapeDtypeStruct((N, HW, Cout), _BF),
        grid=(N, HW // tt),
        in_specs=[pl.BlockSpec((1, Cin, tt), lambda n, t: (n, 0, t)),
                  pl.BlockSpec((Cin, Cout), lambda n, t: (0, 0))],
        out_specs=pl.BlockSpec((1, tt, Cout), lambda n, t: (n, t, 0)),
        compiler_params=pltpu.CompilerParams(
            dimension_semantics=("parallel", "parallel"),
            vmem_limit_bytes=_VMEM_LIMIT),
        cost_estimate=pl.CostEstimate(
            flops=2 * N * HW * Cin * Cout, transcendentals=0,
            bytes_accessed=4 * N * Cin * HW + 2 * N * HW * Cout),
    )(x_flat, wt)


def _lat_chain_kernel(c4_ref, c3_ref, c2_ref, t5_ref, w_ref,
                      o4_ref, o3_ref, o2_ref, ho_ref):
    hb = pl.program_id(1)
    th4, W4 = o4_ref.shape[1], o4_ref.shape[2]
    W3, W2 = o3_ref.shape[2], o2_ref.shape[2]
    t5 = t5_ref[0, pl.ds((th4 // 2) * hb, th4 // 2)].astype(_F32)
    v4 = _latdot(c4_ref[0], w_ref[0:1024]).reshape(o4_ref.shape[1:]) + _nn_up(t5, 2)
    o4_ref[0] = v4.astype(o4_ref.dtype)
    ho_ref[0, 0, 0, 0:W4] = v4[0].astype(ho_ref.dtype)
    ho_ref[0, 0, 1, 0:W4] = v4[-1].astype(ho_ref.dtype)
    v3 = _latdot(c3_ref[0], w_ref[1024:1536]).reshape(o3_ref.shape[1:]) + _nn_up(v4, 2)
    o3_ref[0] = v3.astype(o3_ref.dtype)
    ho_ref[0, 0, 0, W4:W4 + W3] = v3[0].astype(ho_ref.dtype)
    ho_ref[0, 0, 1, W4:W4 + W3] = v3[-1].astype(ho_ref.dtype)
    v2 = _latdot(c2_ref[0], w_ref[1536:1792]).reshape(o2_ref.shape[1:]) + _nn_up(v3, 2)
    o2_ref[0] = v2.astype(o2_ref.dtype)
    ho_ref[0, 0, 0, W4 + W3:W4 + W3 + W2] = v2[0].astype(ho_ref.dtype)
    ho_ref[0, 0, 1, W4 + W3:W4 + W3 + W2] = v2[-1].astype(ho_ref.dtype)


def _lat_chain(c4f, c3f, c2f, t5, wcat, *, th2=16):
    N = c2f.shape[0]
    W2 = t5.shape[2] * 8
    H2 = c2f.shape[2] // W2
    H5, W5 = t5.shape[1], t5.shape[2]
    H4, W4, H3, W3 = H2 // 4, W2 // 4, H2 // 2, W2 // 2
    th4, th3 = th2 // 4, th2 // 2
    nblk = H2 // th2
    Wcat = W4 + W3 + W2
    flops = 2 * N * 256 * (H2 * W2 * 256 + H3 * W3 * 512 + H4 * W4 * 1024)
    out_shapes = [jax.ShapeDtypeStruct((N, H4, W4, 256), _BF),
                  jax.ShapeDtypeStruct((N, H3, W3, 256), _BF),
                  jax.ShapeDtypeStruct((N, H2, W2, 256), _BF),
                  jax.ShapeDtypeStruct((N, nblk, 2, Wcat, 256), _BF)]
    return pl.pallas_call(
        _lat_chain_kernel,
        out_shape=out_shapes,
        grid=(N, nblk),
        in_specs=[
            pl.BlockSpec((1, 1024, th4 * W4), lambda n, h: (n, 0, h)),
            pl.BlockSpec((1, 512, th3 * W3), lambda n, h: (n, 0, h)),
            pl.BlockSpec((1, 256, th2 * W2), lambda n, h: (n, 0, h)),
            pl.BlockSpec((1, H5, W5, 256), lambda n, h: (n, 0, 0, 0)),
            pl.BlockSpec((1792, 256), lambda n, h: (0, 0)),
        ],
        out_specs=[
            pl.BlockSpec((1, th4, W4, 256), lambda n, h: (n, h, 0, 0)),
            pl.BlockSpec((1, th3, W3, 256), lambda n, h: (n, h, 0, 0)),
            pl.BlockSpec((1, th2, W2, 256), lambda n, h: (n, h, 0, 0)),
            pl.BlockSpec((1, 1, 2, Wcat, 256), lambda n, h: (n, h, 0, 0, 0)),
        ],
        compiler_params=pltpu.CompilerParams(
            dimension_semantics=("parallel", "parallel"),
            vmem_limit_bytes=_VMEM_LIMIT),
        cost_estimate=pl.CostEstimate(
            flops=flops, transcendentals=0,
            bytes_accessed=4 * (N * 1024 * H4 * W4 + N * 512 * H3 * W3
                                + N * 256 * H2 * W2)
            + 2 * N * 256 * (H4 * W4 + H3 * W3 + H2 * W2)),
    )(c4f, c3f, c2f, t5, wcat)


_SCALES = (8, 4, 2, 1)
_TH2 = 16


def _fuse_kernel(t5_ref, c4c, c3c, c2c, hp, hn, w_ref, o_ref, s5, s4, s3, s2):
    hb = pl.program_id(1)
    nblk = pl.num_programs(1)
    H5 = t5_ref.shape[1]
    W5 = t5_ref.shape[2]
    W4, W3, W2 = 2 * W5, 4 * W5, 8 * W5
    stks = (s5, s4, s3, s2)
    for idx in range(4):
        scale = _SCALES[idx]
        th = _TH2 // scale
        Br = stks[idx]
        if idx == 0:
            W = W5
            xc = t5_ref[0, pl.ds(th * hb, th)]
            top = t5_ref[0, pl.ds(jnp.maximum(th * hb - 1, 0), 1)][0]
            bot = t5_ref[0, pl.ds(jnp.minimum(th * hb + th, H5 - 1), 1)][0]
        else:
            xc = (c4c, c3c, c2c)[idx - 1][0]
            W = xc.shape[1]
            off = {W4: 0, W3: W4, W2: W4 + W3}[W]
            top = hp[0, 0, 1, off:off + W]
            bot = hn[0, 0, 0, off:off + W]
        T = th * W
        z = jnp.zeros((W, 256), _BF)
        Br[:, 256:512] = xc.reshape(T, 256)
        Br[W:, 0:256] = xc[0:th - 1].reshape(T - W, 256)
        Br[:T - W, 512:768] = xc[1:th].reshape(T - W, 256)
        Br[0:W, 0:256] = jnp.where(hb == 0, z, top.reshape(W, 256))
        Br[T - W:, 512:768] = jnp.where(hb == nblk - 1, z, bot.reshape(W, 256))
        S = jnp.dot(Br[...], w_ref[idx],
                    preferred_element_type=_F32).reshape(th, W, 384)
        zc = jnp.zeros((th, 1, 128), _F32)
        y = (S[:, :, 128:256]
             + jnp.concatenate([zc, S[:, :-1, 0:128]], axis=1)
             + jnp.concatenate([S[:, 1:, 256:384], zc], axis=1))
        up = _nn_up(y, scale)
        hw = up.shape[0] * up.shape[1]
        yt = jnp.transpose(up.reshape(hw, 128))[0:64]
        o_ref[0, 64 * idx:64 * (idx + 1), :] = yt.astype(o_ref.dtype)


def _fused_convs_concat(in5, out4, out3, out2, ho, wstk):
    N, H2, W2, Cb = out2.shape[0], out2.shape[1], out2.shape[2], 64
    H5, W5 = in5.shape[1], in5.shape[2]
    nblk = H2 // _TH2
    Wcat = ho.shape[3]
    flops = sum(2 * 9 * N * (H2 // s) * (W2 // s) * 256 * 64 for s in _SCALES)
    stks = [pltpu.VMEM(((_TH2 // s) * (W2 // s), 768), _BF) for s in _SCALES]
    specs = [
        pl.BlockSpec((1, H5, W5, 256), lambda n, h: (n, 0, 0, 0)),
        pl.BlockSpec((1, _TH2 // 4, W2 // 4, 256), lambda n, h: (n, h, 0, 0)),
        pl.BlockSpec((1, _TH2 // 2, W2 // 2, 256), lambda n, h: (n, h, 0, 0)),
        pl.BlockSpec((1, _TH2, W2, 256), lambda n, h: (n, h, 0, 0)),
        pl.BlockSpec((1, 1, 2, Wcat, 256),
                     lambda n, h: (n, jnp.maximum(h - 1, 0), 0, 0, 0)),
        pl.BlockSpec((1, 1, 2, Wcat, 256),
                     lambda n, h, nb=nblk: (n, jnp.minimum(h + 1, nb - 1), 0, 0, 0)),
        pl.BlockSpec((4, 768, 384), lambda n, h: (0, 0, 0)),
    ]
    return pl.pallas_call(
        _fuse_kernel,
        out_shape=jax.ShapeDtypeStruct((N, 4 * Cb, H2 * W2), _F32),
        grid=(N, nblk),
        in_specs=specs,
        out_specs=pl.BlockSpec((1, 4 * Cb, _TH2 * W2), lambda n, h: (n, 0, h)),
        scratch_shapes=stks,
        compiler_params=pltpu.CompilerParams(
            dimension_semantics=("parallel", "parallel"),
            vmem_limit_bytes=_VMEM_LIMIT),
        cost_estimate=pl.CostEstimate(
            flops=flops, transcendentals=0,
            bytes_accessed=4 * N * H2 * W2 * 4 * Cb
            + 2 * N * (H2 * W2 + (H2 // 2) * (W2 // 2)) * 256),
    )(in5, out4, out3, out2, ho, ho, wstk)


def _mk_conv_w(p):
    wt = jnp.transpose(p, (2, 3, 1, 0))
    wt = jnp.pad(wt, ((0, 0), (0, 0), (0, 0), (0, 64)))
    return jnp.transpose(wt, (0, 2, 1, 3)).reshape(768, 384).astype(_BF)


def kernel(c2, c3, c4, c5, in2, in3, in4, in5, p5, p4, p3, p2):
    N = c2.shape[0]
    w5t = jnp.transpose(in5).astype(_BF)
    wcat = jnp.concatenate([jnp.transpose(in4), jnp.transpose(in3),
                            jnp.transpose(in2)], axis=0).astype(_BF)
    wstk = jnp.stack([_mk_conv_w(p5), _mk_conv_w(p4),
                      _mk_conv_w(p3), _mk_conv_w(p2)])

    c5f = c5.reshape(N, c5.shape[1], -1)
    c4f = c4.reshape(N, c4.shape[1], -1)
    c3f = c3.reshape(N, c3.shape[1], -1)
    c2f = c2.reshape(N, c2.shape[1], -1)

    t5 = _lat_top(c5f, w5t, min(512, c5f.shape[2]))
    t5 = t5.reshape(N, c5.shape[2], c5.shape[3], 256)
    o4, o3, o2, ho = _lat_chain(c4f, c3f, c2f, t5, wcat,
                                th2=min(16, c2.shape[2]))

    fuse = _fused_convs_concat(t5, o4, o3, o2, ho, wstk)
    return fuse.reshape(N, 256, c2.shape[2], c2.shape[3])
